# Initial kernel scaffold; baseline (speedup 1.0000x reference)
import jax
import jax.numpy as jnp
from jax.experimental import pallas as pl


def kernel(x, grid, grid_weight, edge_index, W1, b1, W2, b2, freq, weight):
    raise NotImplementedError("write your pallas kernel here")


# speedup vs baseline: 1.6005x; 1.6005x over previous
import functools

import jax
import jax.numpy as jnp
from jax import lax
from jax.experimental import pallas as pl
from jax.experimental.pallas import tpu as pltpu
from jax.experimental.pallas import tpu_sc as plsc

N = 50000
E = 1600000
NW = 32
EPW = E // NW
C = 80
NCH = EPW // C
SEG = 2 * N
ZB = 10000

TWOPI = 6.283185307179586
INV2PI = 0.15915494309189535
MAGIC = 12582912.0
C3 = -1.0 / 6.0
C5 = 1.0 / 120.0
C7 = -1.0 / 5040.0
C9 = 1.0 / 362880.0
C11 = -1.0 / 39916800.0


def _fc1_body(x_ref, w_ref, b_ref, gw_ref, o_ref):
    acc = jnp.dot(w_ref[...], x_ref[...], preferred_element_type=jnp.float32)
    o_ref[...] = (acc + b_ref[...]) * gw_ref[...]


def _fc2_body(o2_ref, w2_ref, b2_ref, y_ref):
    s = o2_ref[0:32, :] + o2_ref[32:64, :]
    y_ref[...] = jnp.dot(w2_ref[...], s, preferred_element_type=jnp.float32) + b2_ref[...]


def _edge_body(g3_hbm, xs_hbm, src_hbm, dst_hbm, cst_hbm, out_hbm,
               srcv, dstv, gidx, gbuf, geo, cstv, xidx, oidx, xcol, valv,
               zb, outsp, sem_g, sem_x):
    cid = lax.axis_index("c")
    sid = lax.axis_index("s")
    wid = sid * 2 + cid

    pltpu.sync_copy(cst_hbm, cstv)

    @pl.loop(0, ZB // 16)
    def _zero_zb(i):
        zb[pl.ds(i * 16, 16)] = jnp.zeros((16,), jnp.float32)

    @pl.loop(0, SEG // ZB)
    def _zero_outsp(i):
        pltpu.sync_copy(zb, outsp.at[pl.ds(sid * SEG + i * ZB, ZB)])

    plsc.subcore_barrier()

    ebase = wid * EPW

    @pl.loop(0, NCH)
    def _chunk(ci):
        base = ebase + ci * C
        pltpu.sync_copy(src_hbm.at[pl.ds(base, C)], srcv)
        pltpu.sync_copy(dst_hbm.at[pl.ds(base, C)], dstv)
        for j in range(C // 16):
            sl = pl.ds(j * 16, 16)
            s16 = srcv[sl]
            d16 = dstv[sl]
            gidx[0, sl] = s16
            gidx[1, sl] = s16 + N
            gidx[2, sl] = s16 + 2 * N
            gidx[3, sl] = d16
            gidx[4, sl] = d16 + N
            gidx[5, sl] = d16 + 2 * N
        cps = [pltpu.async_copy(g3_hbm.at[gidx.at[k]], gbuf.at[k], sem_g)
               for k in range(6)]
        for cp in cps:
            cp.wait()
        for j in range(C // 16):
            sl = pl.ds(j * 16, 16)
            vx = gbuf[0, sl] - gbuf[3, sl]
            vy = gbuf[1, sl] - gbuf[4, sl]
            vz = gbuf[2, sl] - gbuf[5, sl]
            geo[0, sl] = vx
            geo[1, sl] = vy
            geo[2, sl] = vz
            geo[3, sl] = vx * vx + vy * vy + vz * vz

        @pl.loop(0, 32)
        def _chan(h):
            hN = h * N
            for j in range(C // 16):
                sl = pl.ds(j * 16, 16)
                xidx[sl] = srcv[sl] + hN
                oidx[sl] = dstv[sl] + hN
            pltpu.async_copy(xs_hbm.at[xidx], xcol, sem_x).wait()
            wv = cstv[h, :]
            f0 = cstv[h + 32, :]
            f1 = cstv[h + 64, :]
            f2 = cstv[h + 96, :]
            for j in range(C // 16):
                sl = pl.ds(j * 16, 16)
                vx = geo[0, sl]
                vy = geo[1, sl]
                vz = geo[2, sl]
                d2 = geo[3, sl]
                th = vx * f0 + vy * f1 + vz * f2
                k = (th * INV2PI + MAGIC) - MAGIC
                r = th - k * TWOPI
                s2 = r * r
                p = C11
                p = p * s2 + C9
                p = p * s2 + C7
                p = p * s2 + C5
                p = p * s2 + C3
                p = p * s2 + 1.0
                sinr = r * p
                gauss = jnp.exp(-(wv * d2))
                valv[sl] = gauss * sinr * xcol[sl]
            pltpu.sync_copy(valv, outsp.at[oidx], add=True)

    plsc.subcore_barrier()

    @pl.loop(0, SEG // ZB)
    def _flush(i):
        pltpu.sync_copy(outsp.at[pl.ds(sid * SEG + i * ZB, ZB)], zb)
        pltpu.sync_copy(zb, out_hbm.at[pl.ds(cid * (32 * N) + sid * SEG + i * ZB, ZB)])


_edge_call = pl.kernel(
    _edge_body,
    out_type=jax.ShapeDtypeStruct((2 * 32 * N,), jnp.float32),
    mesh=plsc.VectorSubcoreMesh(core_axis_name="c", subcore_axis_name="s"),
    scratch_types=[
        pltpu.VMEM((C,), jnp.int32),
        pltpu.VMEM((C,), jnp.int32),
        pltpu.VMEM((6, C), jnp.int32),
        pltpu.VMEM((6, C), jnp.float32),
        pltpu.VMEM((4, C), jnp.float32),
        pltpu.VMEM((128, 16), jnp.float32),
        pltpu.VMEM((C,), jnp.int32),
        pltpu.VMEM((C,), jnp.int32),
        pltpu.VMEM((C,), jnp.float32),
        pltpu.VMEM((C,), jnp.float32),
        pltpu.VMEM((ZB,), jnp.float32),
        pltpu.VMEM_SHARED((32 * N,), jnp.float32),
        pltpu.SemaphoreType.DMA,
        pltpu.SemaphoreType.DMA,
    ],
)

NB = 2048
_GRID = (N + NB - 1) // NB

_fc1_call = pl.pallas_call(
    _fc1_body,
    grid=(_GRID,),
    in_specs=[
        pl.BlockSpec((128, NB), lambda i: (0, i)),
        pl.BlockSpec((32, 128), lambda i: (0, 0)),
        pl.BlockSpec((32, 1), lambda i: (0, 0)),
        pl.BlockSpec((1, NB), lambda i: (0, i)),
    ],
    out_specs=pl.BlockSpec((32, NB), lambda i: (0, i)),
    out_shape=jax.ShapeDtypeStruct((32, N), jnp.float32),
)

_fc2_call = pl.pallas_call(
    _fc2_body,
    grid=(_GRID,),
    in_specs=[
        pl.BlockSpec((64, NB), lambda i: (0, i)),
        pl.BlockSpec((128, 32), lambda i: (0, 0)),
        pl.BlockSpec((128, 1), lambda i: (0, 0)),
    ],
    out_specs=pl.BlockSpec((128, NB), lambda i: (0, i)),
    out_shape=jax.ShapeDtypeStruct((128, N), jnp.float32),
)


@jax.jit
def kernel(x, grid, grid_weight, edge_index, W1, b1, W2, b2, freq, weight):
    x2d = x[0]
    gw = grid_weight.reshape(1, N)
    Ch = (weight / jnp.pi) ** 1.5
    W1s = W1 * Ch[:, None]
    b1s = (b1 * Ch).reshape(32, 1)
    xs = _fc1_call(x2d, W1s, b1s, gw)

    g3flat = grid[0].T.reshape(-1)
    src = edge_index[0].astype(jnp.int32)
    dst = edge_index[1].astype(jnp.int32)
    cst = jnp.broadcast_to(
        jnp.concatenate([weight, freq[0], freq[1], freq[2]])[:, None],
        (128, 16)).astype(jnp.float32)

    outflat = _edge_call(g3flat, xs.reshape(-1), src, dst, cst)
    op2 = outflat.reshape(64, N)

    y = _fc2_call(op2, W2, b2.reshape(128, 1))
    return y[None]


# speedup vs baseline: 13.1772x; 8.2330x over previous
import jax
import jax.numpy as jnp
from jax import lax
from jax.experimental import pallas as pl
from jax.experimental.pallas import tpu as pltpu
from jax.experimental.pallas import tpu_sc as plsc

N = 50000
E = 1600000
NW = 32
EPW = E // NW
C = 80
NCH = EPW // C
RPB = 3120
RS = 80

TWOPI = 6.283185307179586
INV2PI = 0.15915494309189535
MAGIC = 12582912.0
C3 = -1.0 / 6.0
C5 = 1.0 / 120.0
C7 = -1.0 / 5040.0
C9 = 1.0 / 362880.0
C11 = -1.0 / 39916800.0

_GDN = lax.GatherDimensionNumbers(offset_dims=(), collapsed_slice_dims=(0,),
                                  start_index_map=(0,))


def _bcast_lane(v, lane):
    return lax.gather(v, lane[:, None], dimension_numbers=_GDN,
                      slice_sizes=(1,),
                      mode=lax.GatherScatterMode.PROMISE_IN_BOUNDS)


def _fc1_body(x_ref, w_ref, b_ref, gw_ref, o_ref):
    acc = lax.dot_general(x_ref[...], w_ref[...],
                          dimension_numbers=(((0,), (1,)), ((), ())),
                          preferred_element_type=jnp.float32)
    o_ref[...] = (acc + b_ref[...]) * gw_ref[...]


def _fc2_body(op_ref, w2_ref, b2_ref, y_ref):
    s = op_ref[0] + op_ref[1]
    y_ref[...] = lax.dot_general(w2_ref[...], s,
                                 dimension_numbers=(((1,), (1,)), ((), ())),
                                 preferred_element_type=jnp.float32) + b2_ref[...]


def _sin(th):
    k = (th * INV2PI + MAGIC) - MAGIC
    r = th - k * TWOPI
    s2 = r * r
    p = C11
    p = p * s2 + C9
    p = p * s2 + C7
    p = p * s2 + C5
    p = p * s2 + C3
    p = p * s2 + 1.0
    return r * p


def _edge_body(gx_hbm, gy_hbm, gz_hbm, xs_hbm, src_hbm, dst_hbm, cst_hbm,
               out_hbm, srcv0, srcv1, dstv0, dstv1, dstcur0, dstcur1, gbuf,
               xb, vb, geo, stage, cstv, outsp,
               sem_i0, sem_i1, sem_g0, sem_g1):
    cid = lax.axis_index("c")
    sid = lax.axis_index("s")
    wid = sid * 2 + cid
    srcvs = (srcv0, srcv1)
    dstvs = (dstv0, dstv1)
    dstcurs = (dstcur0, dstcur1)
    sem_i = (sem_i0, sem_i1)
    sem_g = (sem_g0, sem_g1)

    pltpu.sync_copy(cst_hbm, cstv)

    @pl.loop(0, RS)
    def _zero_stage(i):
        stage[i, 0:16] = jnp.zeros((16,), jnp.float32)
        stage[i, 16:32] = jnp.zeros((16,), jnp.float32)

    @pl.loop(0, RPB // RS)
    def _zero_outsp(i):
        pltpu.sync_copy(stage, outsp.at[pl.ds(sid * RPB + i * RS, RS)])

    @pl.when(sid == 15)
    def _zero_tail():
        pltpu.sync_copy(stage, outsp.at[pl.ds(16 * RPB, RS)])

    plsc.subcore_barrier()

    w0 = cstv[0, :]
    w1 = cstv[1, :]
    f00 = cstv[2, :]
    f01 = cstv[3, :]
    f10 = cstv[4, :]
    f11 = cstv[5, :]
    f20 = cstv[6, :]
    f21 = cstv[7, :]

    def idx_copies(ci, b):
        base = wid * EPW + ci * C
        return [
            pltpu.make_async_copy(src_hbm.at[pl.ds(base, C)], srcvs[b], sem_i[b]),
            pltpu.make_async_copy(dst_hbm.at[pl.ds(base, C)], dstvs[b], sem_i[b]),
        ]

    def gat_copies(b):
        s_idx = srcvs[b]
        d_idx = dstvs[b]
        sem = sem_g[b]
        return [
            pltpu.make_async_copy(gx_hbm.at[s_idx], gbuf.at[b, 0], sem),
            pltpu.make_async_copy(gy_hbm.at[s_idx], gbuf.at[b, 1], sem),
            pltpu.make_async_copy(gz_hbm.at[s_idx], gbuf.at[b, 2], sem),
            pltpu.make_async_copy(gx_hbm.at[d_idx], gbuf.at[b, 3], sem),
            pltpu.make_async_copy(gy_hbm.at[d_idx], gbuf.at[b, 4], sem),
            pltpu.make_async_copy(gz_hbm.at[d_idx], gbuf.at[b, 5], sem),
            pltpu.make_async_copy(xs_hbm.at[s_idx], xb.at[b], sem),
        ]

    def fire_idx(ci, b):
        for cp in idx_copies(ci, b):
            cp.start()

    def wait_idx(ci, b):
        for cp in idx_copies(ci, b):
            cp.wait()

    def fire_gat(b):
        for cp in gat_copies(b):
            cp.start()

    def wait_gat(b):
        for cp in gat_copies(b):
            cp.wait()

    def compute(b):
        dstcur = dstcurs[b]
        for q in range(C // 16):
            sl = pl.ds(q * 16, 16)
            vx = gbuf[b, 0, sl] - gbuf[b, 3, sl]
            vy = gbuf[b, 1, sl] - gbuf[b, 4, sl]
            vz = gbuf[b, 2, sl] - gbuf[b, 5, sl]
            geo[0, q, :] = vx
            geo[1, q, :] = vy
            geo[2, q, :] = vz
            geo[3, q, :] = vx * vx + vy * vy + vz * vz

        @pl.loop(0, C, unroll=4)
        def _edge(e):
            q = e >> 4
            lane = jnp.full((16,), e & 15, jnp.int32)
            bvx = _bcast_lane(geo[0, q, :], lane)
            bvy = _bcast_lane(geo[1, q, :], lane)
            bvz = _bcast_lane(geo[2, q, :], lane)
            bd2 = _bcast_lane(geo[3, q, :], lane)
            th0 = bvx * f00 + bvy * f10 + bvz * f20
            th1 = bvx * f01 + bvy * f11 + bvz * f21
            g0 = jnp.exp(-(w0 * bd2))
            g1 = jnp.exp(-(w1 * bd2))
            vb[b, e, 0:16] = g0 * _sin(th0) * xb[b, e, 0:16]
            vb[b, e, 16:32] = g1 * _sin(th1) * xb[b, e, 16:32]

        pltpu.sync_copy(vb.at[b], outsp.at[dstcur], add=True)

    def step(ci, b, nb):
        wait_idx(ci + 1, nb)
        fire_gat(nb)
        wait_gat(b)
        dstcur = dstcurs[b]
        for q in range(C // 16):
            sl = pl.ds(q * 16, 16)
            dstcur[sl] = dstvs[b][sl]
        @pl.when(ci + 2 < NCH)
        def _prefetch_idx():
            fire_idx(ci + 2, b)
        compute(b)

    fire_idx(0, 0)
    wait_idx(0, 0)
    fire_gat(0)
    fire_idx(1, 1)

    @pl.loop(0, NCH - 1, step=2)
    def _pair(i):
        step(i, 0, 1)
        step(i + 1, 1, 0)

    wait_gat(0)
    dc = dstcurs[0]
    for q in range(C // 16):
        sl = pl.ds(q * 16, 16)
        dc[sl] = dstvs[0][sl]
    compute(0)

    plsc.subcore_barrier()

    @pl.loop(0, RPB // RS)
    def _flush(i):
        pltpu.sync_copy(outsp.at[pl.ds(sid * RPB + i * RS, RS)], stage)
        pltpu.sync_copy(stage, out_hbm.at[cid, pl.ds(sid * RPB + i * RS, RS)])

    @pl.when(sid == 15)
    def _flush_tail():
        pltpu.sync_copy(outsp.at[pl.ds(16 * RPB, RS)], stage)
        pltpu.sync_copy(stage, out_hbm.at[cid, pl.ds(16 * RPB, RS)])


_edge_call = pl.kernel(
    _edge_body,
    out_type=jax.ShapeDtypeStruct((2, N, 32), jnp.float32),
    mesh=plsc.VectorSubcoreMesh(core_axis_name="c", subcore_axis_name="s"),
    compiler_params=pltpu.CompilerParams(use_tc_tiling_on_sc=False),
    scratch_types=[
        pltpu.VMEM((C,), jnp.int32),
        pltpu.VMEM((C,), jnp.int32),
        pltpu.VMEM((C,), jnp.int32),
        pltpu.VMEM((C,), jnp.int32),
        pltpu.VMEM((C,), jnp.int32),
        pltpu.VMEM((C,), jnp.int32),
        pltpu.VMEM((2, 6, C), jnp.float32),
        pltpu.VMEM((2, C, 32), jnp.float32),
        pltpu.VMEM((2, C, 32), jnp.float32),
        pltpu.VMEM((4, C // 16, 16), jnp.float32),
        pltpu.VMEM((RS, 32), jnp.float32),
        pltpu.VMEM((8, 16), jnp.float32),
        pltpu.VMEM_SHARED((N, 32), jnp.float32),
        pltpu.SemaphoreType.DMA,
        pltpu.SemaphoreType.DMA,
        pltpu.SemaphoreType.DMA,
        pltpu.SemaphoreType.DMA,
    ],
)

NB = 2048
_GRID = (N + NB - 1) // NB

_fc1_call = pl.pallas_call(
    _fc1_body,
    grid=(_GRID,),
    in_specs=[
        pl.BlockSpec((128, NB), lambda i: (0, i)),
        pl.BlockSpec((32, 128), lambda i: (0, 0)),
        pl.BlockSpec((1, 32), lambda i: (0, 0)),
        pl.BlockSpec((NB, 1), lambda i: (i, 0)),
    ],
    out_specs=pl.BlockSpec((NB, 32), lambda i: (i, 0)),
    out_shape=jax.ShapeDtypeStruct((N, 32), jnp.float32),
)

_fc2_call = pl.pallas_call(
    _fc2_body,
    grid=(_GRID,),
    in_specs=[
        pl.BlockSpec((2, NB, 32), lambda i: (0, i, 0)),
        pl.BlockSpec((128, 32), lambda i: (0, 0)),
        pl.BlockSpec((128, 1), lambda i: (0, 0)),
    ],
    out_specs=pl.BlockSpec((128, NB), lambda i: (0, i)),
    out_shape=jax.ShapeDtypeStruct((128, N), jnp.float32),
)


@jax.jit
def kernel(x, grid, grid_weight, edge_index, W1, b1, W2, b2, freq, weight):
    x2d = x[0]
    gw = grid_weight.reshape(N, 1)
    Ch = (weight / jnp.pi) ** 1.5
    W1s = W1 * Ch[:, None]
    b1s = (b1 * Ch).reshape(1, 32)
    xs = _fc1_call(x2d, W1s, b1s, gw)

    gx = grid[0, :, 0]
    gy = grid[0, :, 1]
    gz = grid[0, :, 2]
    src2 = edge_index[0].astype(jnp.int32)
    dst2 = edge_index[1].astype(jnp.int32)
    cst = jnp.stack([weight[:16], weight[16:],
                     freq[0, :16], freq[0, 16:],
                     freq[1, :16], freq[1, 16:],
                     freq[2, :16], freq[2, 16:]]).astype(jnp.float32)

    outp = _edge_call(gx, gy, gz, xs, src2, dst2, cst)

    y = _fc2_call(outp, W2, b2.reshape(128, 1))
    return y[None]


# speedup vs baseline: 20.0439x; 1.5211x over previous
import jax
import jax.numpy as jnp
from jax import lax
from jax.experimental import pallas as pl
from jax.experimental.pallas import tpu as pltpu
from jax.experimental.pallas import tpu_sc as plsc

N = 50000
E = 1600000
NW = 32
EPW = E // NW
C = 80
NCH = EPW // C
RPB = 3120
RS = 40

TWOPI = 6.283185307179586
INV2PI = 0.15915494309189535
MAGIC = 12582912.0
C3 = -1.0 / 6.0
C5 = 1.0 / 120.0
C7 = -1.0 / 5040.0
C9 = 1.0 / 362880.0
C11 = -1.0 / 39916800.0

_GDN = lax.GatherDimensionNumbers(offset_dims=(), collapsed_slice_dims=(0,),
                                  start_index_map=(0,))


def _bcast_lane(v, lane):
    return lax.gather(v, lane[:, None], dimension_numbers=_GDN,
                      slice_sizes=(1,),
                      mode=lax.GatherScatterMode.PROMISE_IN_BOUNDS)


def _fc1_body(x_ref, w_ref, b_ref, gw_ref, g3_ref, fr_ref, p_ref, q_ref, ca_ref, sa_ref):
    acc = lax.dot_general(x_ref[...], w_ref[...],
                          dimension_numbers=(((0,), (1,)), ((), ())),
                          precision=lax.Precision.HIGHEST,
                          preferred_element_type=jnp.float32)
    xs = (acc + b_ref[...]) * gw_ref[...]
    g3 = g3_ref[...]
    fr = fr_ref[...]
    a = (g3[:, 0:1] * fr[0:1, :] + g3[:, 1:2] * fr[1:2, :]
         + g3[:, 2:3] * fr[2:3, :])
    sa = jnp.sin(a)
    ca = jnp.cos(a)
    p_ref[...] = xs * sa
    q_ref[...] = xs * ca
    ca_ref[...] = ca
    sa_ref[...] = sa


def _fc2_body(op_ref, w2_ref, b2_ref, y_ref):
    s = op_ref[0] + op_ref[1]
    y_ref[...] = lax.dot_general(w2_ref[...], s,
                                 dimension_numbers=(((1,), (1,)), ((), ())),
                                 preferred_element_type=jnp.float32) + b2_ref[...]


def _sin(th):
    k = (th * INV2PI + MAGIC) - MAGIC
    r = th - k * TWOPI
    s2 = r * r
    p = C11
    p = p * s2 + C9
    p = p * s2 + C7
    p = p * s2 + C5
    p = p * s2 + C3
    p = p * s2 + 1.0
    return r * p


def _edge_body(gx_hbm, gy_hbm, gz_hbm, p_hbm, q_hbm, ca_hbm, sa_hbm, src_hbm, dst_hbm, cst_hbm,
               out_hbm, srcv0, srcv1, dstv0, dstv1, dstcur0, dstcur1, gbuf,
               pb, qb, cab, sab, vb, geo, stage, cstv, outsp,
               sem_i0, sem_i1, sem_g0, sem_g1):
    cid = lax.axis_index("c")
    sid = lax.axis_index("s")
    wid = sid * 2 + cid
    srcvs = (srcv0, srcv1)
    dstvs = (dstv0, dstv1)
    dstcurs = (dstcur0, dstcur1)
    sem_i = (sem_i0, sem_i1)
    sem_g = (sem_g0, sem_g1)

    pltpu.sync_copy(cst_hbm, cstv)

    @pl.loop(0, RS)
    def _zero_stage(i):
        stage[i, 0:16] = jnp.zeros((16,), jnp.float32)
        stage[i, 16:32] = jnp.zeros((16,), jnp.float32)

    @pl.loop(0, RPB // RS)
    def _zero_outsp(i):
        pltpu.sync_copy(stage, outsp.at[pl.ds(sid * RPB + i * RS, RS)])

    @pl.when(sid == 15)
    def _zero_tail():
        @pl.loop(0, (N - 16 * RPB) // RS)
        def _zt(i):
            pltpu.sync_copy(stage, outsp.at[pl.ds(16 * RPB + i * RS, RS)])

    plsc.subcore_barrier()

    w0 = cstv[0, :]
    w1 = cstv[1, :]

    def idx_copies(ci, b):
        base = wid * EPW + ci * C
        return [
            pltpu.make_async_copy(src_hbm.at[pl.ds(base, C)], srcvs[b], sem_i[b]),
            pltpu.make_async_copy(dst_hbm.at[pl.ds(base, C)], dstvs[b], sem_i[b]),
        ]

    def gat_copies(b):
        s_idx = srcvs[b]
        d_idx = dstvs[b]
        sem = sem_g[b]
        return [
            pltpu.make_async_copy(gx_hbm.at[s_idx], gbuf.at[b, 0], sem),
            pltpu.make_async_copy(gy_hbm.at[s_idx], gbuf.at[b, 1], sem),
            pltpu.make_async_copy(gz_hbm.at[s_idx], gbuf.at[b, 2], sem),
            pltpu.make_async_copy(gx_hbm.at[d_idx], gbuf.at[b, 3], sem),
            pltpu.make_async_copy(gy_hbm.at[d_idx], gbuf.at[b, 4], sem),
            pltpu.make_async_copy(gz_hbm.at[d_idx], gbuf.at[b, 5], sem),
            pltpu.make_async_copy(p_hbm.at[s_idx], pb.at[b], sem),
            pltpu.make_async_copy(q_hbm.at[s_idx], qb.at[b], sem),
            pltpu.make_async_copy(ca_hbm.at[d_idx], cab.at[b], sem),
            pltpu.make_async_copy(sa_hbm.at[d_idx], sab.at[b], sem),
        ]

    def fire_idx(ci, b):
        for cp in idx_copies(ci, b):
            cp.start()

    def wait_idx(ci, b):
        for cp in idx_copies(ci, b):
            cp.wait()

    def fire_gat(b):
        for cp in gat_copies(b):
            cp.start()

    def wait_gat(b):
        for cp in gat_copies(b):
            cp.wait()

    def compute(b):
        dstcur = dstcurs[b]
        for q in range(C // 16):
            sl = pl.ds(q * 16, 16)
            vx = gbuf[b, 0, sl] - gbuf[b, 3, sl]
            vy = gbuf[b, 1, sl] - gbuf[b, 4, sl]
            vz = gbuf[b, 2, sl] - gbuf[b, 5, sl]
            geo[0, q, :] = vx * vx + vy * vy + vz * vz

        @pl.loop(0, C, unroll=4)
        def _edge(e):
            q = e >> 4
            lane = jnp.full((16,), e & 15, jnp.int32)
            bd2 = _bcast_lane(geo[0, q, :], lane)
            g0 = jnp.exp(-(w0 * bd2))
            g1 = jnp.exp(-(w1 * bd2))
            p0 = pb[b, e, 0:16]
            p1 = pb[b, e, 16:32]
            q0 = qb[b, e, 0:16]
            q1 = qb[b, e, 16:32]
            ca0 = cab[b, e, 0:16]
            ca1 = cab[b, e, 16:32]
            sa0 = sab[b, e, 0:16]
            sa1 = sab[b, e, 16:32]
            vb[b, e, 0:16] = g0 * (p0 * ca0 - q0 * sa0)
            vb[b, e, 16:32] = g1 * (p1 * ca1 - q1 * sa1)

        pltpu.sync_copy(vb.at[b], outsp.at[dstcur], add=True)

    def step(ci, b, nb):
        wait_idx(ci + 1, nb)
        fire_gat(nb)
        wait_gat(b)
        dstcur = dstcurs[b]
        for q in range(C // 16):
            sl = pl.ds(q * 16, 16)
            dstcur[sl] = dstvs[b][sl]
        @pl.when(ci + 2 < NCH)
        def _prefetch_idx():
            fire_idx(ci + 2, b)
        compute(b)

    fire_idx(0, 0)
    wait_idx(0, 0)
    fire_gat(0)
    fire_idx(1, 1)

    @pl.loop(0, NCH - 1, step=2)
    def _pair(i):
        step(i, 0, 1)
        step(i + 1, 1, 0)

    wait_gat(0)
    dc = dstcurs[0]
    for q in range(C // 16):
        sl = pl.ds(q * 16, 16)
        dc[sl] = dstvs[0][sl]
    compute(0)

    plsc.subcore_barrier()

    @pl.loop(0, RPB // RS)
    def _flush(i):
        pltpu.sync_copy(outsp.at[pl.ds(sid * RPB + i * RS, RS)], stage)
        pltpu.sync_copy(stage, out_hbm.at[cid, pl.ds(sid * RPB + i * RS, RS)])

    @pl.when(sid == 15)
    def _flush_tail():
        @pl.loop(0, (N - 16 * RPB) // RS)
        def _ft(i):
            pltpu.sync_copy(outsp.at[pl.ds(16 * RPB + i * RS, RS)], stage)
            pltpu.sync_copy(stage, out_hbm.at[cid, pl.ds(16 * RPB + i * RS, RS)])


_edge_call = pl.kernel(
    _edge_body,
    out_type=jax.ShapeDtypeStruct((2, N, 32), jnp.float32),
    mesh=plsc.VectorSubcoreMesh(core_axis_name="c", subcore_axis_name="s"),
    compiler_params=pltpu.CompilerParams(use_tc_tiling_on_sc=False),
    scratch_types=[
        pltpu.VMEM((C,), jnp.int32),
        pltpu.VMEM((C,), jnp.int32),
        pltpu.VMEM((C,), jnp.int32),
        pltpu.VMEM((C,), jnp.int32),
        pltpu.VMEM((C,), jnp.int32),
        pltpu.VMEM((C,), jnp.int32),
        pltpu.VMEM((2, 6, C), jnp.float32),
        pltpu.VMEM((2, C, 32), jnp.float32),
        pltpu.VMEM((2, C, 32), jnp.float32),
        pltpu.VMEM((2, C, 32), jnp.float32),
        pltpu.VMEM((2, C, 32), jnp.float32),
        pltpu.VMEM((2, C, 32), jnp.float32),
        pltpu.VMEM((1, C // 16, 16), jnp.float32),
        pltpu.VMEM((RS, 32), jnp.float32),
        pltpu.VMEM((2, 16), jnp.float32),
        pltpu.VMEM_SHARED((N, 32), jnp.float32),
        pltpu.SemaphoreType.DMA,
        pltpu.SemaphoreType.DMA,
        pltpu.SemaphoreType.DMA,
        pltpu.SemaphoreType.DMA,
    ],
)

NB = 2048
_GRID = (N + NB - 1) // NB

_fc1_call = pl.pallas_call(
    _fc1_body,
    grid=(_GRID,),
    in_specs=[
        pl.BlockSpec((128, NB), lambda i: (0, i)),
        pl.BlockSpec((32, 128), lambda i: (0, 0)),
        pl.BlockSpec((1, 32), lambda i: (0, 0)),
        pl.BlockSpec((NB, 1), lambda i: (i, 0)),
        pl.BlockSpec((NB, 3), lambda i: (i, 0)),
        pl.BlockSpec((3, 32), lambda i: (0, 0)),
    ],
    out_specs=[
        pl.BlockSpec((NB, 32), lambda i: (i, 0)),
        pl.BlockSpec((NB, 32), lambda i: (i, 0)),
        pl.BlockSpec((NB, 32), lambda i: (i, 0)),
        pl.BlockSpec((NB, 32), lambda i: (i, 0)),
    ],
    out_shape=[
        jax.ShapeDtypeStruct((N, 32), jnp.float32),
        jax.ShapeDtypeStruct((N, 32), jnp.float32),
        jax.ShapeDtypeStruct((N, 32), jnp.float32),
        jax.ShapeDtypeStruct((N, 32), jnp.float32),
    ],
)

_fc2_call = pl.pallas_call(
    _fc2_body,
    grid=(_GRID,),
    in_specs=[
        pl.BlockSpec((2, NB, 32), lambda i: (0, i, 0)),
        pl.BlockSpec((128, 32), lambda i: (0, 0)),
        pl.BlockSpec((128, 1), lambda i: (0, 0)),
    ],
    out_specs=pl.BlockSpec((128, NB), lambda i: (0, i)),
    out_shape=jax.ShapeDtypeStruct((128, N), jnp.float32),
)


@jax.jit
def kernel(x, grid, grid_weight, edge_index, W1, b1, W2, b2, freq, weight):
    x2d = x[0]
    gw = grid_weight.reshape(N, 1)
    Ch = (weight / jnp.pi) ** 1.5
    W1s = W1 * Ch[:, None]
    b1s = (b1 * Ch).reshape(1, 32)
    p_t, q_t, ca_t, sa_t = _fc1_call(x2d, W1s, b1s, gw, grid[0], freq)

    gx = grid[0, :, 0]
    gy = grid[0, :, 1]
    gz = grid[0, :, 2]
    src2 = edge_index[0].astype(jnp.int32)
    dst2 = edge_index[1].astype(jnp.int32)
    cst = jnp.stack([weight[:16], weight[16:]]).astype(jnp.float32)

    outp = _edge_call(gx, gy, gz, p_t, q_t, ca_t, sa_t, src2, dst2, cst)

    y = _fc2_call(outp, W2, b2.reshape(128, 1))
    return y[None]
